# Initial kernel scaffold; baseline (speedup 1.0000x reference)
import jax
import jax.numpy as jnp
from jax.experimental import pallas as pl


def kernel(z_h, z_l, edge_index_h_l, We1, be1, We2, be2, Ww1, bw1, Ww2, bw2, Wn1, bn1, Wn2, bn2):
    raise NotImplementedError("write your pallas kernel here")


# speedup vs baseline: 2.7004x; 2.7004x over previous
import functools

import jax
import jax.numpy as jnp
from jax import lax
from jax.experimental import pallas as pl
from jax.experimental.pallas import tpu as pltpu
from jax.experimental.pallas import tpu_sc as plsc

N = 10000
E = 320000
FEAT = 128
MSG = 128

NC, NS = 2, 16
NW = NC * NS
EPW = E // NW
CHUNK = 80
NPAD = 10240
NPS = NPAD // NS

def _sc_mesh():
    return plsc.VectorSubcoreMesh(core_axis_name="c", subcore_axis_name="s",
                                  num_cores=NC, num_subcores=NS)


def _sc_gather(z_h, z_l, src, tgt):
    @functools.partial(
        pl.kernel,
        out_type=(jax.ShapeDtypeStruct((E, FEAT), jnp.float32),
                  jax.ShapeDtypeStruct((E, FEAT), jnp.float32)),
        mesh=_sc_mesh(),
        scratch_types=[pltpu.VMEM((CHUNK,), jnp.int32),
                       pltpu.VMEM((CHUNK,), jnp.int32),
                       pltpu.VMEM((CHUNK, FEAT), jnp.float32),
                       pltpu.VMEM((CHUNK, FEAT), jnp.float32),
                       pltpu.SemaphoreType.DMA,
                       pltpu.SemaphoreType.DMA],
    )
    def gather_kernel(zh_hbm, zl_hbm, src_hbm, tgt_hbm, outs_hbm, outt_hbm,
                      idxs_v, idxt_v, rows_s, rows_t, sem_s, sem_t):
        wid = lax.axis_index("s") * NC + lax.axis_index("c")
        base = wid * EPW

        @pl.loop(0, EPW // CHUNK)
        def _(i):
            off = base + i * CHUNK
            pltpu.sync_copy(src_hbm.at[pl.ds(off, CHUNK)], idxs_v)
            pltpu.sync_copy(tgt_hbm.at[pl.ds(off, CHUNK)], idxt_v)
            cs = pltpu.async_copy(zh_hbm.at[idxs_v], rows_s, sem_s)
            ct = pltpu.async_copy(zl_hbm.at[idxt_v], rows_t, sem_t)
            cs.wait()
            ct.wait()
            pltpu.sync_copy(rows_s, outs_hbm.at[pl.ds(off, CHUNK)])
            pltpu.sync_copy(rows_t, outt_hbm.at[pl.ds(off, CHUNK)])

    return gather_kernel(z_h, z_l, src, tgt)


def _sc_scatter(msg, tgt, zeros):
    @functools.partial(
        pl.kernel,
        out_type=jax.ShapeDtypeStruct((NC, NPAD, MSG), jnp.float32),
        mesh=_sc_mesh(),
        scratch_types=[pltpu.VMEM((1, CHUNK), jnp.int32),
                       pltpu.VMEM((CHUNK, MSG), jnp.float32),
                       pltpu.VMEM_SHARED((NPAD, MSG), jnp.float32),
                       pltpu.SemaphoreType.DMA],
    )
    def scatter_kernel(msg_hbm, tgt_hbm, zero_hbm, out_hbm,
                       idx_v, msg_v, acc_sh, sem):
        cid = lax.axis_index("c")
        sid = lax.axis_index("s")
        pltpu.sync_copy(zero_hbm.at[pl.ds(sid * NPS, NPS)],
                        acc_sh.at[pl.ds(sid * NPS, NPS)])
        plsc.subcore_barrier()
        base = (sid * NC + cid) * EPW

        @pl.loop(0, EPW // CHUNK)
        def _(i):
            off = base + i * CHUNK
            pltpu.sync_copy(tgt_hbm.at[pl.ds(off, CHUNK)], idx_v.at[0])
            pltpu.sync_copy(msg_hbm.at[pl.ds(off, CHUNK)], msg_v)
            pltpu.sync_copy(msg_v, acc_sh.at[idx_v.at[0]], add=True)

        plsc.subcore_barrier()
        pltpu.sync_copy(acc_sh.at[pl.ds(sid * NPS, NPS)],
                        out_hbm.at[cid].at[pl.ds(sid * NPS, NPS)])

    return scatter_kernel(msg, tgt, zeros)


def _dot(a, b):
    return lax.dot_general(a, b, (((1,), (0,)), ((), ())),
                           preferred_element_type=jnp.float32)


def _edge_body(gs_ref, gt_ref, we1_ref, be1_ref, ww1_ref, bw1_ref,
               we2_ref, be2_ref, ww2_ref, bw2_ref, out_ref):
    s = gs_ref[...]
    t = gt_ref[...]

    def col(x, j):
        return x[:, j:j + 1]

    d0 = col(s, 0) - col(t, 0)
    d1 = col(s, 1) - col(t, 1)
    d2 = col(s, 2) - col(t, 2)
    dist = d0 * d0 + d1 * d1 + d2 * d2
    s3, s4, s5 = col(s, 3), col(s, 4), col(s, 5)
    t3, t4, t5 = col(t, 3), col(t, 4), col(t, 5)
    c0 = s4 * t5 - s5 * t4
    c1 = s5 * t3 - s3 * t5
    c2 = s3 * t4 - s4 * t3
    absc = jnp.sqrt(c0 * c0 + c1 * c1 + c2 * c2)
    feats = (d0, d1, d2, dist, c0, c1, c2, absc)

    we1 = we1_ref[...]
    ww1 = ww1_ref[...]
    h = _dot(s, we1[0:FEAT]) + _dot(t, we1[FEAT:2 * FEAT]) + be1_ref[...]
    hw = _dot(s, ww1[0:FEAT]) + _dot(t, ww1[FEAT:2 * FEAT]) + bw1_ref[...]
    for j, f in enumerate(feats):
        h = h + f * we1[2 * FEAT + j:2 * FEAT + j + 1, :]
        hw = hw + f * ww1[2 * FEAT + j:2 * FEAT + j + 1, :]
    h = jnp.maximum(h, 0.0)
    hw = jnp.maximum(hw, 0.0)

    wlog = _dot(hw, ww2_ref[...]) + bw2_ref[...]
    w = 1.0 / (1.0 + jnp.exp(-wlog))

    out_ref[...] = w * (_dot(h, we2_ref[...]) + be2_ref[...])


def _tc_edge(gs, gt, We1, be1, Ww1, bw1, We2, be2, Ww2, bw2):
    B = 2000

    def wspec(shape):
        return pl.BlockSpec(shape, lambda i: (0,) * len(shape))

    return pl.pallas_call(
        _edge_body,
        grid=(E // B,),
        in_specs=[pl.BlockSpec((B, FEAT), lambda i: (i, 0)),
                  pl.BlockSpec((B, FEAT), lambda i: (i, 0)),
                  wspec((2 * FEAT + 8, 64)), wspec((1, 64)),
                  wspec((2 * FEAT + 8, 64)), wspec((1, 64)),
                  wspec((64, 128)), wspec((1, 128)),
                  wspec((64, 1)), wspec((1, 1))],
        out_specs=pl.BlockSpec((B, MSG), lambda i: (i, 0)),
        out_shape=jax.ShapeDtypeStruct((E, MSG), jnp.float32),
    )(gs, gt, We1, be1, Ww1, bw1, We2, be2, Ww2, bw2)


def _node_body(zl_ref, agg_ref, wn1a_ref, wn1b_ref, bn1_ref, wn2_ref, bn2_ref,
               out_ref):
    a = agg_ref[...]
    agg = a[0] + a[1]
    h = jnp.maximum(_dot(zl_ref[...], wn1a_ref[...]) + _dot(agg, wn1b_ref[...])
                    + bn1_ref[...], 0.0)
    out_ref[...] = _dot(h, wn2_ref[...]) + bn2_ref[...]


def _tc_node(z_l, agg, Wn1a, Wn1b, bn1, Wn2, bn2):
    B = 1000

    def wspec(shape):
        return pl.BlockSpec(shape, lambda i: (0,) * len(shape))

    return pl.pallas_call(
        _node_body,
        grid=(N // B,),
        in_specs=[pl.BlockSpec((B, FEAT), lambda i: (i, 0)),
                  pl.BlockSpec((NC, B, MSG), lambda i: (0, i, 0)),
                  wspec((FEAT, 64)), wspec((FEAT, 64)), wspec((1, 64)),
                  wspec((64, FEAT)), wspec((1, FEAT))],
        out_specs=pl.BlockSpec((B, FEAT), lambda i: (i, 0)),
        out_shape=jax.ShapeDtypeStruct((N, FEAT), jnp.float32),
    )(z_l, agg, Wn1a, Wn1b, bn1, Wn2, bn2)


@jax.jit
def kernel(z_h, z_l, edge_index_h_l, We1, be1, We2, be2, Ww1, bw1, Ww2, bw2,
           Wn1, bn1, Wn2, bn2):
    ei = edge_index_h_l.astype(jnp.int32)
    src = ei[0]
    tgt = ei[1]

    gs, gt = _sc_gather(z_h, z_l, src, tgt)
    msg = _tc_edge(gs, gt,
                   We1, be1.reshape(1, 64),
                   Ww1, bw1.reshape(1, 64),
                   We2, be2.reshape(1, 128),
                   Ww2, bw2.reshape(1, 1))
    zeros = jnp.zeros((NPAD, MSG), jnp.float32)
    agg = _sc_scatter(msg, tgt, zeros)
    return _tc_node(z_l, agg, Wn1[:FEAT], Wn1[FEAT:], bn1.reshape(1, 64),
                    Wn2, bn2.reshape(1, 128))


# speedup vs baseline: 2.7062x; 1.0021x over previous
import functools

import jax
import jax.numpy as jnp
from jax import lax
from jax.experimental import pallas as pl
from jax.experimental.pallas import tpu as pltpu
from jax.experimental.pallas import tpu_sc as plsc

N = 10000
E = 320000
FEAT = 128
MSG = 128

NC, NS = 2, 16
NW = NC * NS
EPW = E // NW
CHUNK = 80
NPAD = 10240
NPS = NPAD // NS

def _sc_mesh():
    return plsc.VectorSubcoreMesh(core_axis_name="c", subcore_axis_name="s",
                                  num_cores=NC, num_subcores=NS)


def _sc_gather(z_h, z_l, src, tgt):
    @functools.partial(
        pl.kernel,
        out_type=(jax.ShapeDtypeStruct((E, FEAT), jnp.float32),
                  jax.ShapeDtypeStruct((E, FEAT), jnp.float32)),
        mesh=_sc_mesh(),
        scratch_types=[pltpu.VMEM((CHUNK,), jnp.int32),
                       pltpu.VMEM((CHUNK,), jnp.int32),
                       pltpu.VMEM((CHUNK, FEAT), jnp.float32),
                       pltpu.VMEM((CHUNK, FEAT), jnp.float32),
                       pltpu.SemaphoreType.DMA,
                       pltpu.SemaphoreType.DMA],
    )
    def gather_kernel(zh_hbm, zl_hbm, src_hbm, tgt_hbm, outs_hbm, outt_hbm,
                      idxs_v, idxt_v, rows_s, rows_t, sem_s, sem_t):
        wid = lax.axis_index("s") * NC + lax.axis_index("c")
        base = wid * EPW

        @pl.loop(0, EPW // CHUNK)
        def _(i):
            off = base + i * CHUNK
            pltpu.sync_copy(src_hbm.at[pl.ds(off, CHUNK)], idxs_v)
            pltpu.sync_copy(tgt_hbm.at[pl.ds(off, CHUNK)], idxt_v)
            cs = pltpu.async_copy(zh_hbm.at[idxs_v], rows_s, sem_s)
            ct = pltpu.async_copy(zl_hbm.at[idxt_v], rows_t, sem_t)
            cs.wait()
            ct.wait()
            pltpu.sync_copy(rows_s, outs_hbm.at[pl.ds(off, CHUNK)])
            pltpu.sync_copy(rows_t, outt_hbm.at[pl.ds(off, CHUNK)])

    return gather_kernel(z_h, z_l, src, tgt)


def _sc_scatter(msg, tgt, zeros):
    @functools.partial(
        pl.kernel,
        out_type=jax.ShapeDtypeStruct((NC, NPAD, MSG), jnp.float32),
        mesh=_sc_mesh(),
        scratch_types=[pltpu.VMEM((1, CHUNK), jnp.int32),
                       pltpu.VMEM((CHUNK, MSG), jnp.float32),
                       pltpu.VMEM_SHARED((NPAD, MSG), jnp.float32),
                       pltpu.SemaphoreType.DMA],
    )
    def scatter_kernel(msg_hbm, tgt_hbm, zero_hbm, out_hbm,
                       idx_v, msg_v, acc_sh, sem):
        cid = lax.axis_index("c")
        sid = lax.axis_index("s")
        pltpu.sync_copy(zero_hbm.at[pl.ds(sid * NPS, NPS)],
                        acc_sh.at[pl.ds(sid * NPS, NPS)])
        plsc.subcore_barrier()
        base = (sid * NC + cid) * EPW

        @pl.loop(0, EPW // CHUNK)
        def _(i):
            off = base + i * CHUNK
            pltpu.sync_copy(tgt_hbm.at[pl.ds(off, CHUNK)], idx_v.at[0])
            pltpu.sync_copy(msg_hbm.at[pl.ds(off, CHUNK)], msg_v)
            pltpu.sync_copy(msg_v, acc_sh.at[idx_v.at[0]], add=True)

        plsc.subcore_barrier()
        pltpu.sync_copy(acc_sh.at[pl.ds(sid * NPS, NPS)],
                        out_hbm.at[cid].at[pl.ds(sid * NPS, NPS)])

    return scatter_kernel(msg, tgt, zeros)


def _dot(a, b):
    return lax.dot_general(a, b, (((1,), (0,)), ((), ())),
                           preferred_element_type=jnp.float32)


def _edge_body(gs_ref, gt_ref, we1_ref, be1_ref, ww1_ref, bw1_ref,
               we2_ref, be2_ref, ww2_ref, bw2_ref, out_ref):
    s = gs_ref[...]
    t = gt_ref[...]

    def col(x, j):
        return x[:, j:j + 1]

    d0 = col(s, 0) - col(t, 0)
    d1 = col(s, 1) - col(t, 1)
    d2 = col(s, 2) - col(t, 2)
    dist = d0 * d0 + d1 * d1 + d2 * d2
    s3, s4, s5 = col(s, 3), col(s, 4), col(s, 5)
    t3, t4, t5 = col(t, 3), col(t, 4), col(t, 5)
    c0 = s4 * t5 - s5 * t4
    c1 = s5 * t3 - s3 * t5
    c2 = s3 * t4 - s4 * t3
    absc = jnp.sqrt(c0 * c0 + c1 * c1 + c2 * c2)
    feats = (d0, d1, d2, dist, c0, c1, c2, absc)

    we1 = we1_ref[...]
    ww1 = ww1_ref[...]
    sb = s.astype(jnp.bfloat16)
    tb = t.astype(jnp.bfloat16)
    we1b = we1.astype(jnp.bfloat16)
    ww1b = ww1.astype(jnp.bfloat16)
    h = (_dot(sb, we1b[0:FEAT]) + _dot(tb, we1b[FEAT:2 * FEAT])
         + be1_ref[...])
    hw = (_dot(sb, ww1b[0:FEAT]) + _dot(tb, ww1b[FEAT:2 * FEAT])
          + bw1_ref[...])
    for j, f in enumerate(feats):
        h = h + f * we1[2 * FEAT + j:2 * FEAT + j + 1, :]
        hw = hw + f * ww1[2 * FEAT + j:2 * FEAT + j + 1, :]
    h = jnp.maximum(h, 0.0)
    hw = jnp.maximum(hw, 0.0)

    wlog = _dot(hw.astype(jnp.bfloat16),
                ww2_ref[...].astype(jnp.bfloat16)) + bw2_ref[...]
    w = 1.0 / (1.0 + jnp.exp(-wlog))

    out_ref[...] = w * (_dot(h.astype(jnp.bfloat16),
                             we2_ref[...].astype(jnp.bfloat16)) + be2_ref[...])


def _tc_edge(gs, gt, We1, be1, Ww1, bw1, We2, be2, Ww2, bw2):
    B = 2000

    def wspec(shape):
        return pl.BlockSpec(shape, lambda i: (0,) * len(shape))

    return pl.pallas_call(
        _edge_body,
        grid=(E // B,),
        in_specs=[pl.BlockSpec((B, FEAT), lambda i: (i, 0)),
                  pl.BlockSpec((B, FEAT), lambda i: (i, 0)),
                  wspec((2 * FEAT + 8, 64)), wspec((1, 64)),
                  wspec((2 * FEAT + 8, 64)), wspec((1, 64)),
                  wspec((64, 128)), wspec((1, 128)),
                  wspec((64, 1)), wspec((1, 1))],
        out_specs=pl.BlockSpec((B, MSG), lambda i: (i, 0)),
        out_shape=jax.ShapeDtypeStruct((E, MSG), jnp.float32),
    )(gs, gt, We1, be1, Ww1, bw1, We2, be2, Ww2, bw2)


def _node_body(zl_ref, agg_ref, wn1a_ref, wn1b_ref, bn1_ref, wn2_ref, bn2_ref,
               out_ref):
    a = agg_ref[...]
    agg = a[0] + a[1]
    h = jnp.maximum(_dot(zl_ref[...], wn1a_ref[...]) + _dot(agg, wn1b_ref[...])
                    + bn1_ref[...], 0.0)
    out_ref[...] = _dot(h, wn2_ref[...]) + bn2_ref[...]


def _tc_node(z_l, agg, Wn1a, Wn1b, bn1, Wn2, bn2):
    B = 1000

    def wspec(shape):
        return pl.BlockSpec(shape, lambda i: (0,) * len(shape))

    return pl.pallas_call(
        _node_body,
        grid=(N // B,),
        in_specs=[pl.BlockSpec((B, FEAT), lambda i: (i, 0)),
                  pl.BlockSpec((NC, B, MSG), lambda i: (0, i, 0)),
                  wspec((FEAT, 64)), wspec((FEAT, 64)), wspec((1, 64)),
                  wspec((64, FEAT)), wspec((1, FEAT))],
        out_specs=pl.BlockSpec((B, FEAT), lambda i: (i, 0)),
        out_shape=jax.ShapeDtypeStruct((N, FEAT), jnp.float32),
    )(z_l, agg, Wn1a, Wn1b, bn1, Wn2, bn2)


@jax.jit
def kernel(z_h, z_l, edge_index_h_l, We1, be1, We2, be2, Ww1, bw1, Ww2, bw2,
           Wn1, bn1, Wn2, bn2):
    ei = edge_index_h_l.astype(jnp.int32)
    src = ei[0]
    tgt = ei[1]

    gs, gt = _sc_gather(z_h, z_l, src, tgt)
    msg = _tc_edge(gs, gt,
                   We1, be1.reshape(1, 64),
                   Ww1, bw1.reshape(1, 64),
                   We2, be2.reshape(1, 128),
                   Ww2, bw2.reshape(1, 1))
    zeros = jnp.zeros((NPAD, MSG), jnp.float32)
    agg = _sc_scatter(msg, tgt, zeros)
    return _tc_node(z_l, agg, Wn1[:FEAT], Wn1[FEAT:], bn1.reshape(1, 64),
                    Wn2, bn2.reshape(1, 128))


# speedup vs baseline: 4.4303x; 1.6371x over previous
import functools

import jax
import jax.numpy as jnp
from jax import lax
from jax.experimental import pallas as pl
from jax.experimental.pallas import tpu as pltpu
from jax.experimental.pallas import tpu_sc as plsc

N = 10000
E = 320000
FEAT = 128
HID = 64
MSG = 128

NC, NS = 2, 16
NW = NC * NS
EPW = E // NW
CHUNK = 80
NPAD = 10240
NPS = NPAD // NS

def _sc_mesh():
    return plsc.VectorSubcoreMesh(core_axis_name="c", subcore_axis_name="s",
                                  num_cores=NC, num_subcores=NS)


def _sc_gather(z_h, z_l, src, tgt):
    @functools.partial(
        pl.kernel,
        out_type=(jax.ShapeDtypeStruct((E, FEAT), jnp.float32),
                  jax.ShapeDtypeStruct((E, FEAT), jnp.float32)),
        mesh=_sc_mesh(),
        scratch_types=[pltpu.VMEM((CHUNK,), jnp.int32),
                       pltpu.VMEM((CHUNK,), jnp.int32),
                       pltpu.VMEM((CHUNK, FEAT), jnp.float32),
                       pltpu.VMEM((CHUNK, FEAT), jnp.float32),
                       pltpu.SemaphoreType.DMA,
                       pltpu.SemaphoreType.DMA],
    )
    def gather_kernel(zh_hbm, zl_hbm, src_hbm, tgt_hbm, outs_hbm, outt_hbm,
                      idxs_v, idxt_v, rows_s, rows_t, sem_s, sem_t):
        wid = lax.axis_index("s") * NC + lax.axis_index("c")
        base = wid * EPW

        @pl.loop(0, EPW // CHUNK)
        def _(i):
            off = base + i * CHUNK
            pltpu.sync_copy(src_hbm.at[pl.ds(off, CHUNK)], idxs_v)
            pltpu.sync_copy(tgt_hbm.at[pl.ds(off, CHUNK)], idxt_v)
            cs = pltpu.async_copy(zh_hbm.at[idxs_v], rows_s, sem_s)
            ct = pltpu.async_copy(zl_hbm.at[idxt_v], rows_t, sem_t)
            cs.wait()
            ct.wait()
            pltpu.sync_copy(rows_s, outs_hbm.at[pl.ds(off, CHUNK)])
            pltpu.sync_copy(rows_t, outt_hbm.at[pl.ds(off, CHUNK)])

    return gather_kernel(z_h, z_l, src, tgt)


def _sc_scatter(msg, tgt, zeros):
    @functools.partial(
        pl.kernel,
        out_type=jax.ShapeDtypeStruct((NC, NPAD, MSG), jnp.float32),
        mesh=_sc_mesh(),
        scratch_types=[pltpu.VMEM((1, CHUNK), jnp.int32),
                       pltpu.VMEM((CHUNK, MSG), jnp.float32),
                       pltpu.VMEM_SHARED((NPAD, MSG), jnp.float32),
                       pltpu.SemaphoreType.DMA],
    )
    def scatter_kernel(msg_hbm, tgt_hbm, zero_hbm, out_hbm,
                       idx_v, msg_v, acc_sh, sem):
        cid = lax.axis_index("c")
        sid = lax.axis_index("s")
        pltpu.sync_copy(zero_hbm.at[pl.ds(sid * NPS, NPS)],
                        acc_sh.at[pl.ds(sid * NPS, NPS)])
        plsc.subcore_barrier()
        base = (sid * NC + cid) * EPW

        @pl.loop(0, EPW // CHUNK)
        def _(i):
            off = base + i * CHUNK
            pltpu.sync_copy(tgt_hbm.at[pl.ds(off, CHUNK)], idx_v.at[0])
            pltpu.sync_copy(msg_hbm.at[pl.ds(off, CHUNK)], msg_v)
            pltpu.sync_copy(msg_v, acc_sh.at[idx_v.at[0]], add=True)

        plsc.subcore_barrier()
        pltpu.sync_copy(acc_sh.at[pl.ds(sid * NPS, NPS)],
                        out_hbm.at[cid].at[pl.ds(sid * NPS, NPS)])

    return scatter_kernel(msg, tgt, zeros)


def _dot(a, b):
    return lax.dot_general(a, b, (((1,), (0,)), ((), ())),
                           preferred_element_type=jnp.float32)


def _edge_body(gs_ref, gt_ref, w1cat_ref, b1cat_ref, wecat_t_ref,
               we2_ref, be2_ref, ww2_ref, bw2_ref, out_ref):
    s = gs_ref[...]
    t = gt_ref[...]

    st8 = jnp.concatenate([s[:, 0:8], t[:, 0:8]], axis=1)
    st8_t = st8.T
    s_t = st8_t[0:8]
    t_t = st8_t[8:16]
    d = s_t[0:3] - t_t[0:3]
    dist = jnp.sum(d * d, axis=0, keepdims=True)
    a = s_t[3:6]
    b = t_t[3:6]
    c0 = a[1:2] * b[2:3] - a[2:3] * b[1:2]
    c1 = a[2:3] * b[0:1] - a[0:1] * b[2:3]
    c2 = a[0:1] * b[1:2] - a[1:2] * b[0:1]
    c = jnp.concatenate([c0, c1, c2], axis=0)
    absc = jnp.sqrt(jnp.sum(c * c, axis=0, keepdims=True))
    e8_t = jnp.concatenate([d, dist, c, absc], axis=0)
    ep_t = _dot(wecat_t_ref[...], e8_t)
    ep = ep_t.T

    x = jnp.concatenate([s, t], axis=1).astype(jnp.bfloat16)
    hcat = jnp.maximum(_dot(x, w1cat_ref[...]) + ep + b1cat_ref[...], 0.0)
    h = hcat[:, 0:HID]
    hw = hcat[:, HID:2 * HID]

    wlog = _dot(hw.astype(jnp.bfloat16), ww2_ref[...]) + bw2_ref[...]
    w = 1.0 / (1.0 + jnp.exp(-wlog))

    out_ref[...] = w * (_dot(h.astype(jnp.bfloat16), we2_ref[...])
                        + be2_ref[...])


def _tc_edge(gs, gt, W1cat, b1cat, WecatT, We2, be2, Ww2, bw2):
    B = 2000

    def wspec(shape):
        return pl.BlockSpec(shape, lambda i: (0,) * len(shape))

    return pl.pallas_call(
        _edge_body,
        grid=(E // B,),
        in_specs=[pl.BlockSpec((B, FEAT), lambda i: (i, 0)),
                  pl.BlockSpec((B, FEAT), lambda i: (i, 0)),
                  wspec((2 * FEAT, 2 * HID)), wspec((1, 2 * HID)),
                  wspec((2 * HID, 8)),
                  wspec((HID, 128)), wspec((1, 128)),
                  wspec((HID, 1)), wspec((1, 1))],
        out_specs=pl.BlockSpec((B, MSG), lambda i: (i, 0)),
        out_shape=jax.ShapeDtypeStruct((E, MSG), jnp.float32),
    )(gs, gt, W1cat, b1cat, WecatT, We2, be2, Ww2, bw2)


def _node_body(zl_ref, agg_ref, wn1a_ref, wn1b_ref, bn1_ref, wn2_ref, bn2_ref,
               out_ref):
    a = agg_ref[...]
    agg = a[0] + a[1]
    h = jnp.maximum(_dot(zl_ref[...], wn1a_ref[...]) + _dot(agg, wn1b_ref[...])
                    + bn1_ref[...], 0.0)
    out_ref[...] = _dot(h, wn2_ref[...]) + bn2_ref[...]


def _tc_node(z_l, agg, Wn1a, Wn1b, bn1, Wn2, bn2):
    B = 1000

    def wspec(shape):
        return pl.BlockSpec(shape, lambda i: (0,) * len(shape))

    return pl.pallas_call(
        _node_body,
        grid=(N // B,),
        in_specs=[pl.BlockSpec((B, FEAT), lambda i: (i, 0)),
                  pl.BlockSpec((NC, B, MSG), lambda i: (0, i, 0)),
                  wspec((FEAT, 64)), wspec((FEAT, 64)), wspec((1, 64)),
                  wspec((64, FEAT)), wspec((1, FEAT))],
        out_specs=pl.BlockSpec((B, FEAT), lambda i: (i, 0)),
        out_shape=jax.ShapeDtypeStruct((N, FEAT), jnp.float32),
    )(z_l, agg, Wn1a, Wn1b, bn1, Wn2, bn2)


@jax.jit
def kernel(z_h, z_l, edge_index_h_l, We1, be1, We2, be2, Ww1, bw1, Ww2, bw2,
           Wn1, bn1, Wn2, bn2):
    ei = edge_index_h_l.astype(jnp.int32)
    src = ei[0]
    tgt = ei[1]

    gs, gt = _sc_gather(z_h, z_l, src, tgt)
    W1cat = jnp.concatenate([We1[:2 * FEAT], Ww1[:2 * FEAT]],
                            axis=1).astype(jnp.bfloat16)
    b1cat = jnp.concatenate([be1, bw1]).reshape(1, 2 * HID)
    WecatT = jnp.concatenate([We1[2 * FEAT:], Ww1[2 * FEAT:]], axis=1).T
    msg = _tc_edge(gs, gt, W1cat, b1cat, WecatT,
                   We2.astype(jnp.bfloat16), be2.reshape(1, 128),
                   Ww2.astype(jnp.bfloat16), bw2.reshape(1, 1))
    zeros = jnp.zeros((NPAD, MSG), jnp.float32)
    agg = _sc_scatter(msg, tgt, zeros)
    return _tc_node(z_l, agg, Wn1[:FEAT], Wn1[FEAT:], bn1.reshape(1, 64),
                    Wn2, bn2.reshape(1, 128))


# speedup vs baseline: 5.5934x; 1.2625x over previous
import functools

import jax
import jax.numpy as jnp
from jax import lax
from jax.experimental import pallas as pl
from jax.experimental.pallas import tpu as pltpu
from jax.experimental.pallas import tpu_sc as plsc

N = 10000
E = 320000
FEAT = 128
HID = 64
MSG = 128

NC, NS = 2, 16
NW = NC * NS
EPW = E // NW
CHUNK = 80
NPAD = 10240
NPS = NPAD // NS

def _sc_mesh():
    return plsc.VectorSubcoreMesh(core_axis_name="c", subcore_axis_name="s",
                                  num_cores=NC, num_subcores=NS)


def _sc_gather(z_h, z_l, src, tgt, n_edges):
    epw = n_edges // NW

    @functools.partial(
        pl.kernel,
        out_type=(jax.ShapeDtypeStruct((n_edges, FEAT), jnp.float32),
                  jax.ShapeDtypeStruct((n_edges, FEAT), jnp.float32)),
        mesh=_sc_mesh(),
        scratch_types=[pltpu.VMEM((CHUNK,), jnp.int32),
                       pltpu.VMEM((CHUNK,), jnp.int32),
                       pltpu.VMEM((CHUNK, FEAT), jnp.float32),
                       pltpu.VMEM((CHUNK, FEAT), jnp.float32),
                       pltpu.SemaphoreType.DMA,
                       pltpu.SemaphoreType.DMA],
    )
    def gather_kernel(zh_hbm, zl_hbm, src_hbm, tgt_hbm, outs_hbm, outt_hbm,
                      idxs_v, idxt_v, rows_s, rows_t, sem_s, sem_t):
        wid = lax.axis_index("s") * NC + lax.axis_index("c")
        base = wid * epw

        @pl.loop(0, epw // CHUNK)
        def _(i):
            off = base + i * CHUNK
            pltpu.sync_copy(src_hbm.at[pl.ds(off, CHUNK)], idxs_v)
            pltpu.sync_copy(tgt_hbm.at[pl.ds(off, CHUNK)], idxt_v)
            cs = pltpu.async_copy(zh_hbm.at[idxs_v], rows_s, sem_s)
            ct = pltpu.async_copy(zl_hbm.at[idxt_v], rows_t, sem_t)
            cs.wait()
            ct.wait()
            pltpu.sync_copy(rows_s, outs_hbm.at[pl.ds(off, CHUNK)])
            pltpu.sync_copy(rows_t, outt_hbm.at[pl.ds(off, CHUNK)])

    return gather_kernel(z_h, z_l, src, tgt)


def _sc_scatter(msg, tgt, zeros, n_edges):
    epw = n_edges // NW

    @functools.partial(
        pl.kernel,
        out_type=jax.ShapeDtypeStruct((NC, NPAD, MSG), jnp.float32),
        mesh=_sc_mesh(),
        scratch_types=[pltpu.VMEM((1, CHUNK), jnp.int32),
                       pltpu.VMEM((CHUNK, MSG), jnp.float32),
                       pltpu.VMEM_SHARED((NPAD, MSG), jnp.float32),
                       pltpu.SemaphoreType.DMA],
    )
    def scatter_kernel(msg_hbm, tgt_hbm, zero_hbm, out_hbm,
                       idx_v, msg_v, acc_sh, sem):
        cid = lax.axis_index("c")
        sid = lax.axis_index("s")
        pltpu.sync_copy(zero_hbm.at[pl.ds(sid * NPS, NPS)],
                        acc_sh.at[pl.ds(sid * NPS, NPS)])
        plsc.subcore_barrier()
        base = (sid * NC + cid) * epw

        @pl.loop(0, epw // CHUNK)
        def _(i):
            off = base + i * CHUNK
            pltpu.sync_copy(tgt_hbm.at[pl.ds(off, CHUNK)], idx_v.at[0])
            pltpu.sync_copy(msg_hbm.at[pl.ds(off, CHUNK)], msg_v)
            pltpu.sync_copy(msg_v, acc_sh.at[idx_v.at[0]], add=True)

        plsc.subcore_barrier()
        pltpu.sync_copy(acc_sh.at[pl.ds(sid * NPS, NPS)],
                        out_hbm.at[cid].at[pl.ds(sid * NPS, NPS)])

    return scatter_kernel(msg, tgt, zeros)


def _dot(a, b):
    return lax.dot_general(a, b, (((1,), (0,)), ((), ())),
                           preferred_element_type=jnp.float32)


def _edge_body(gs_ref, gt_ref, w1cat_ref, b1cat_ref, wecat_t_ref,
               we2_ref, be2_ref, ww2_ref, bw2_ref, out_ref):
    s = gs_ref[...]
    t = gt_ref[...]

    st8 = jnp.concatenate([s[:, 0:8], t[:, 0:8]], axis=1)
    st8_t = st8.T
    s_t = st8_t[0:8]
    t_t = st8_t[8:16]
    d = s_t[0:3] - t_t[0:3]
    dist = jnp.sum(d * d, axis=0, keepdims=True)
    a = s_t[3:6]
    b = t_t[3:6]
    c0 = a[1:2] * b[2:3] - a[2:3] * b[1:2]
    c1 = a[2:3] * b[0:1] - a[0:1] * b[2:3]
    c2 = a[0:1] * b[1:2] - a[1:2] * b[0:1]
    c = jnp.concatenate([c0, c1, c2], axis=0)
    absc = jnp.sqrt(jnp.sum(c * c, axis=0, keepdims=True))
    e8_t = jnp.concatenate([d, dist, c, absc], axis=0)
    ep_t = _dot(wecat_t_ref[...], e8_t)
    ep = ep_t.T

    x = jnp.concatenate([s, t], axis=1).astype(jnp.bfloat16)
    hcat = jnp.maximum(_dot(x, w1cat_ref[...]) + ep + b1cat_ref[...], 0.0)
    h = hcat[:, 0:HID]
    hw = hcat[:, HID:2 * HID]

    wlog = _dot(hw.astype(jnp.bfloat16), ww2_ref[...]) + bw2_ref[...]
    w = 1.0 / (1.0 + jnp.exp(-wlog))

    out_ref[...] = w * (_dot(h.astype(jnp.bfloat16), we2_ref[...])
                        + be2_ref[...])


def _tc_edge(gs, gt, W1cat, b1cat, WecatT, We2, be2, Ww2, bw2, n_edges, B):
    def wspec(shape):
        return pl.BlockSpec(shape, lambda i: (0,) * len(shape))

    return pl.pallas_call(
        _edge_body,
        grid=(n_edges // B,),
        in_specs=[pl.BlockSpec((B, FEAT), lambda i: (i, 0)),
                  pl.BlockSpec((B, FEAT), lambda i: (i, 0)),
                  wspec((2 * FEAT, 2 * HID)), wspec((1, 2 * HID)),
                  wspec((2 * HID, 8)),
                  wspec((HID, 128)), wspec((1, 128)),
                  wspec((HID, 1)), wspec((1, 1))],
        out_specs=pl.BlockSpec((B, MSG), lambda i: (i, 0)),
        out_shape=jax.ShapeDtypeStruct((n_edges, MSG), jnp.float32),
    )(gs, gt, W1cat, b1cat, WecatT, We2, be2, Ww2, bw2)


def _node_body(zl_ref, agg_ref, wn1a_ref, wn1b_ref, bn1_ref, wn2_ref, bn2_ref,
               out_ref):
    a = agg_ref[...]
    agg = jnp.sum(a, axis=0)
    h = jnp.maximum(_dot(zl_ref[...], wn1a_ref[...]) + _dot(agg, wn1b_ref[...])
                    + bn1_ref[...], 0.0)
    out_ref[...] = _dot(h, wn2_ref[...]) + bn2_ref[...]


def _tc_node(z_l, agg, Wn1a, Wn1b, bn1, Wn2, bn2):
    B = 1000
    P = agg.shape[0]

    def wspec(shape):
        return pl.BlockSpec(shape, lambda i: (0,) * len(shape))

    return pl.pallas_call(
        _node_body,
        grid=(N // B,),
        in_specs=[pl.BlockSpec((B, FEAT), lambda i: (i, 0)),
                  pl.BlockSpec((P, B, MSG), lambda i: (0, i, 0)),
                  wspec((FEAT, 64)), wspec((FEAT, 64)), wspec((1, 64)),
                  wspec((64, FEAT)), wspec((1, FEAT))],
        out_specs=pl.BlockSpec((B, FEAT), lambda i: (i, 0)),
        out_shape=jax.ShapeDtypeStruct((N, FEAT), jnp.float32),
    )(z_l, agg, Wn1a, Wn1b, bn1, Wn2, bn2)


@jax.jit
def kernel(z_h, z_l, edge_index_h_l, We1, be1, We2, be2, Ww1, bw1, Ww2, bw2,
           Wn1, bn1, Wn2, bn2):
    ei = edge_index_h_l.astype(jnp.int32)
    src = ei[0]
    tgt = ei[1]

    W1cat = jnp.concatenate([We1[:2 * FEAT], Ww1[:2 * FEAT]],
                            axis=1).astype(jnp.bfloat16)
    b1cat = jnp.concatenate([be1, bw1]).reshape(1, 2 * HID)
    WecatT = jnp.concatenate([We1[2 * FEAT:], Ww1[2 * FEAT:]], axis=1).T
    zeros = jnp.zeros((NPAD, MSG), jnp.float32)

    slices = ((0, 62 * NW * CHUNK, 1984), (62 * NW * CHUNK, 63 * NW * CHUNK,
                                           2016))
    aggs = []
    for off, ne, blk in slices:
        s_s = lax.dynamic_slice_in_dim(src, off, ne)
        t_s = lax.dynamic_slice_in_dim(tgt, off, ne)
        gs, gt = _sc_gather(z_h, z_l, s_s, t_s, ne)
        msg = _tc_edge(gs, gt, W1cat, b1cat, WecatT,
                       We2.astype(jnp.bfloat16), be2.reshape(1, 128),
                       Ww2.astype(jnp.bfloat16), bw2.reshape(1, 1), ne, blk)
        aggs.append(_sc_scatter(msg, t_s, zeros, ne))

    agg = jnp.concatenate(aggs, axis=0)
    return _tc_node(z_l, agg, Wn1[:FEAT], Wn1[FEAT:], bn1.reshape(1, 64),
                    Wn2, bn2.reshape(1, 128))
